# Initial kernel scaffold; baseline (speedup 1.0000x reference)
import jax
import jax.numpy as jnp
from jax.experimental import pallas as pl


def kernel(x, edge_index, Wi, bi, W1a, b1a, W2a, b2a, W1b, W2b):
    raise NotImplementedError("write your pallas kernel here")


# speedup vs baseline: 11.9079x; 11.9079x over previous
import functools
import jax
import jax.numpy as jnp
from jax import lax
from jax.experimental import pallas as pl
from jax.experimental.pallas import tpu as pltpu
from jax.experimental.pallas import tpu_sc as plsc

_N = 10000
_D = 128
_E = 320000
_SCALE = 0.8
_NC = 2
_NS = 16
_CH = 128
_NCHUNK = 80
_NPAD = 10240
_EPAD = _NC * _NS * _NCHUNK * _CH
_RPT = _NPAD // _NS
_BLK = 1024
_GRID = _NPAD // _BLK

_mesh = plsc.VectorSubcoreMesh(core_axis_name="c", subcore_axis_name="s")


def _deg_body(dst_hbm, ones_hbm, out_hbm, dstv, onesv, deg_sh):
    c = lax.axis_index("c")
    s = lax.axis_index("s")
    wid = c * _NS + s
    tb = s * _RPT
    pltpu.sync_copy(dst_hbm.at[wid], dstv)
    pltpu.sync_copy(ones_hbm, onesv)
    for k in range(_RPT // _CH):
        pltpu.sync_copy(onesv, deg_sh.at[pl.ds(tb + k * _CH, _CH)])
    plsc.subcore_barrier()

    def body(j, carry):
        pltpu.sync_copy(onesv, deg_sh.at[dstv.at[j]], add=True)
        return carry

    lax.fori_loop(0, _NCHUNK, body, 0)
    plsc.subcore_barrier()
    pltpu.sync_copy(deg_sh.at[pl.ds(tb, _RPT)],
                    out_hbm.at[pl.ds(c * _NPAD + tb, _RPT)])


_deg_call = pl.kernel(
    _deg_body,
    out_type=jax.ShapeDtypeStruct((_NC * _NPAD, 8), jnp.float32),
    mesh=_mesh,
    scratch_types=[
        pltpu.VMEM((_NCHUNK, _CH), jnp.int32),
        pltpu.VMEM((_CH, 8), jnp.float32),
        pltpu.VMEM_SHARED((_NPAD, 8), jnp.float32),
    ],
)


def _scat_body(src_hbm, dst_hbm, hs_hbm, out_hbm, srcv, dstv, rows, acc_sh, sem):
    c = lax.axis_index("c")
    s = lax.axis_index("s")
    wid = c * _NS + s
    tb = s * _RPT
    pltpu.sync_copy(src_hbm.at[wid], srcv)
    pltpu.sync_copy(dst_hbm.at[wid], dstv)
    pltpu.sync_copy(hs_hbm.at[pl.ds(tb, _RPT)], acc_sh.at[pl.ds(tb, _RPT)])
    plsc.subcore_barrier()

    def body(j, carry):
        pltpu.async_copy(hs_hbm.at[srcv.at[j]], rows, sem).wait()
        pltpu.sync_copy(rows, acc_sh.at[dstv.at[j]], add=True)
        return carry

    lax.fori_loop(0, _NCHUNK, body, 0)
    plsc.subcore_barrier()
    pltpu.sync_copy(acc_sh.at[pl.ds(tb, _RPT)],
                    out_hbm.at[pl.ds(c * _NPAD + tb, _RPT)])


_scat_call = pl.kernel(
    _scat_body,
    out_type=jax.ShapeDtypeStruct((_NC * _NPAD, _D), jnp.float32),
    mesh=_mesh,
    scratch_types=[
        pltpu.VMEM((_NCHUNK, _CH), jnp.int32),
        pltpu.VMEM((_NCHUNK, _CH), jnp.int32),
        pltpu.VMEM((_CH, _D), jnp.float32),
        pltpu.VMEM_SHARED((_NPAD, _D), jnp.float32),
        pltpu.SemaphoreType.DMA,
    ],
)


def _dense_body(x_ref, wi_ref, bi_ref, w2a_ref, b2a_ref, w2b_ref,
                d0_ref, d1_ref, z_ref, hs_ref, dinv_ref, z2_ref):
    xb = x_ref[...]
    z = jnp.dot(xb, wi_ref[...], preferred_element_type=jnp.float32) + bi_ref[...]
    z_ref[...] = z
    p = jnp.dot(z, w2a_ref[...], preferred_element_type=jnp.float32) + b2a_ref[...]
    nrm = jnp.sqrt(jnp.sum(p * p, axis=1, keepdims=True))
    h = p / jnp.maximum(nrm, 1e-12) * _SCALE
    deg = d0_ref[...] + d1_ref[...] - 1.0
    dinv = 1.0 / jnp.sqrt(deg)
    dinv_ref[...] = dinv
    rows = pl.program_id(0) * _BLK + lax.broadcasted_iota(jnp.int32, (_BLK, 1), 0)
    hs_ref[...] = jnp.where(rows < _N, h * dinv[:, :1], 0.0)
    q = jnp.dot(xb, w2b_ref[...], preferred_element_type=jnp.float32)
    nq = jnp.sqrt(jnp.sum(q * q, axis=1, keepdims=True))
    z2_ref[...] = q / jnp.maximum(nq, 1e-12) * _SCALE


_dense_call = pl.pallas_call(
    _dense_body,
    grid=(_GRID,),
    in_specs=[
        pl.BlockSpec((_BLK, _D), lambda i: (i, 0)),
        pl.BlockSpec((_D, _D), lambda i: (0, 0)),
        pl.BlockSpec((1, _D), lambda i: (0, 0)),
        pl.BlockSpec((_D, _D), lambda i: (0, 0)),
        pl.BlockSpec((1, _D), lambda i: (0, 0)),
        pl.BlockSpec((_D, _D), lambda i: (0, 0)),
        pl.BlockSpec((_BLK, 8), lambda i: (i, 0)),
        pl.BlockSpec((_BLK, 8), lambda i: (i, 0)),
    ],
    out_specs=[
        pl.BlockSpec((_BLK, _D), lambda i: (i, 0)),
        pl.BlockSpec((_BLK, _D), lambda i: (i, 0)),
        pl.BlockSpec((_BLK, 8), lambda i: (i, 0)),
        pl.BlockSpec((_BLK, _D), lambda i: (i, 0)),
    ],
    out_shape=[
        jax.ShapeDtypeStruct((_NPAD, _D), jnp.float32),
        jax.ShapeDtypeStruct((_NPAD, _D), jnp.float32),
        jax.ShapeDtypeStruct((_NPAD, 8), jnp.float32),
        jax.ShapeDtypeStruct((_NPAD, _D), jnp.float32),
    ],
)


def _comb_body(z_ref, hs_ref, dinv_ref, z2_ref, a0_ref, a1_ref, out_ref):
    z1 = dinv_ref[...][:, :1] * (a0_ref[...] + a1_ref[...] - hs_ref[...])
    out_ref[:, 0:_D] = z_ref[...]
    out_ref[:, _D:2 * _D] = z1
    out_ref[:, 2 * _D:3 * _D] = z2_ref[...]


_comb_call = pl.pallas_call(
    _comb_body,
    grid=(_GRID,),
    in_specs=[
        pl.BlockSpec((_BLK, _D), lambda i: (i, 0)),
        pl.BlockSpec((_BLK, _D), lambda i: (i, 0)),
        pl.BlockSpec((_BLK, 8), lambda i: (i, 0)),
        pl.BlockSpec((_BLK, _D), lambda i: (i, 0)),
        pl.BlockSpec((_BLK, _D), lambda i: (i, 0)),
        pl.BlockSpec((_BLK, _D), lambda i: (i, 0)),
    ],
    out_specs=pl.BlockSpec((_BLK, 3 * _D), lambda i: (i, 0)),
    out_shape=jax.ShapeDtypeStruct((_NPAD, 3 * _D), jnp.float32),
)


@jax.jit
def kernel(x, edge_index, Wi, bi, W1a, b1a, W2a, b2a, W1b, W2b):
    src = edge_index[0]
    dst = edge_index[1]
    pad = _EPAD - _E
    srcp = jnp.concatenate([src, jnp.full((pad,), _N, jnp.int32)]
                           ).reshape(_NC * _NS, _NCHUNK, _CH)
    dstp = jnp.concatenate([dst, jnp.full((pad,), _N, jnp.int32)]
                           ).reshape(_NC * _NS, _NCHUNK, _CH)
    xp = jnp.pad(x, ((0, _NPAD - _N), (0, 0)))
    ones8 = jnp.ones((_CH, 8), jnp.float32)

    deg = _deg_call(dstp, ones8)
    z, hs, dinv, z2 = _dense_call(xp, Wi, bi.reshape(1, _D), W2a,
                                  b2a.reshape(1, _D), W2b,
                                  deg[:_NPAD], deg[_NPAD:])
    acc = _scat_call(srcp, dstp, hs)
    z0 = _comb_call(z, hs, dinv, z2, acc[:_NPAD], acc[_NPAD:])
    return z0[:_N]
